# baseline (speedup 1.0000x reference)
import jax
import jax.numpy as jnp
from jax import lax
from jax.experimental import pallas as pl
from jax.experimental.pallas import tpu as pltpu
from jax.experimental.pallas import tpu_sc as plsc

_D = 64
_ROWS = 16384
_COLS = 50
_B = _ROWS * _COLS
_L = 16
_NC, _NS = 2, 16
_NW = _NC * _NS
_CHUNK = 128
_BPW = _B // _NW
_CPW = _BPW // _CHUNK
_SCALE = 8.0


def _sc_body(idx_hbm, table_hbm, out_hbm, idx_v, rows_v, sem):
    wid = lax.axis_index("s") * _NC + lax.axis_index("c")
    pltpu.sync_copy(idx_hbm.at[pl.ds(wid * _CPW, _CPW)], idx_v)
    base = wid * _BPW

    def chunk_body(c, carry):
        pltpu.async_copy(table_hbm.at[idx_v.at[c]], rows_v, sem).wait()

        def row_body(i, carry2):
            for j in range(_D // _L):
                sl = pl.ds(j * _L, _L)
                rows_v[i, sl] = rows_v[i, sl] * _SCALE
            return carry2

        lax.fori_loop(0, _CHUNK, row_body, 0)
        pltpu.sync_copy(rows_v, out_hbm.at[pl.ds(base + c * _CHUNK, _CHUNK)])
        return carry

    lax.fori_loop(0, _CPW, chunk_body, 0)


def kernel(x, lut):
    idx = x.reshape(_B // _CHUNK, _CHUNK).astype(jnp.int32)
    mesh = plsc.VectorSubcoreMesh(core_axis_name="c", subcore_axis_name="s")
    sc_call = pl.kernel(
        _sc_body,
        mesh=mesh,
        out_type=jax.ShapeDtypeStruct((_B, _D), jnp.float32),
        scratch_types=[
            pltpu.VMEM((_CPW, _CHUNK), jnp.int32),
            pltpu.VMEM((_CHUNK, _D), jnp.float32),
            pltpu.SemaphoreType.DMA,
        ],
        compiler_params=pltpu.CompilerParams(use_tc_tiling_on_sc=False),
    )
    out = sc_call(idx, lut)
    return out.reshape(_ROWS, _COLS, _D)


# speedup vs baseline: 1.2112x; 1.2112x over previous
import jax
import jax.numpy as jnp
from jax import lax
from jax.experimental import pallas as pl
from jax.experimental.pallas import tpu as pltpu
from jax.experimental.pallas import tpu_sc as plsc

_D = 64
_ROWS = 16384
_COLS = 50
_B = _ROWS * _COLS
_L = 16
_NC, _NS = 2, 16
_NW = _NC * _NS
_CHUNK = 128
_BPW = _B // _NW
_CPW = _BPW // _CHUNK
_SCALE = 8.0
_NBUF = 8
_LOOK = 4


def _sc_body(idx_hbm, table_hbm, out_hbm, idx_v, rows_v, g_sem, w_sem):
    wid = lax.axis_index("s") * _NC + lax.axis_index("c")
    pltpu.sync_copy(idx_hbm.at[pl.ds(wid * _CPW, _CPW)], idx_v)
    base = wid * _BPW

    def fire_gather(c, b):
        pltpu.async_copy(table_hbm.at[idx_v.at[c]], rows_v.at[b], g_sem.at[b])

    def wait_gather(c, b):
        pltpu.make_async_copy(
            table_hbm.at[idx_v.at[c]], rows_v.at[b], g_sem.at[b]).wait()

    def fire_write(c, b):
        pltpu.async_copy(
            rows_v.at[b], out_hbm.at[pl.ds(base + c * _CHUNK, _CHUNK)],
            w_sem.at[b])

    def wait_write(b):
        pltpu.make_async_copy(
            rows_v.at[b], out_hbm.at[pl.ds(base, _CHUNK)], w_sem.at[b]).wait()

    for b in range(_LOOK):
        fire_gather(b, b)

    def group(g, carry):
        for b in range(_NBUF):
            c = g * _NBUF + b
            bb = (b + _LOOK) % _NBUF
            wait_gather(c, b)

            def row4(i, carry2, _b=b):
                for j in range(4):
                    r = i * 4 + j
                    for k in range(_D // _L):
                        sl = pl.ds(k * _L, _L)
                        rows_v[_b, r, sl] = rows_v[_b, r, sl] * _SCALE
                return carry2

            lax.fori_loop(0, _CHUNK // 4, row4, 0)
            fire_write(c, b)

            @pl.when(c + _LOOK < _CPW)
            def _(c=c, bb=bb):
                @pl.when(c >= _LOOK)
                def _():
                    wait_write(bb)
                fire_gather(c + _LOOK, bb)

        return carry

    lax.fori_loop(0, _CPW // _NBUF, group, 0)
    for b in range(_NBUF):
        wait_write(b)


def kernel(x, lut):
    idx = x.reshape(_B // _CHUNK, _CHUNK).astype(jnp.int32)
    mesh = plsc.VectorSubcoreMesh(core_axis_name="c", subcore_axis_name="s")
    sc_call = pl.kernel(
        _sc_body,
        mesh=mesh,
        out_type=jax.ShapeDtypeStruct((_B, _D), jnp.float32),
        scratch_types=[
            pltpu.VMEM((_CPW, _CHUNK), jnp.int32),
            pltpu.VMEM((_NBUF, _CHUNK, _D), jnp.float32),
            pltpu.SemaphoreType.DMA((_NBUF,)),
            pltpu.SemaphoreType.DMA((_NBUF,)),
        ],
        compiler_params=pltpu.CompilerParams(use_tc_tiling_on_sc=False),
    )
    out = sc_call(idx, lut)
    return out.reshape(_ROWS, _COLS, _D)
